# baseline (speedup 1.0000x reference)
import jax
import jax.numpy as jnp
from jax.experimental import pallas as pl
from jax.experimental.pallas import tpu as pltpu

B = 8
H_ENC = 1024
H_DEC = 1024
TOTAL = 16384
SEG = TOTAL // B
BLK = 512
SPS = SEG // BLK


def _attn_body(hs_ref, enc_ref, wd_ref, we_ref, v_ref, out_ref):
    i = pl.program_id(0)
    b = i // SPS
    j = i % SPS
    h = hs_ref[pl.ds(b, 1), :]
    hproj = jnp.dot(h, wd_ref[...], preferred_element_type=jnp.float32)
    x = jnp.dot(enc_ref[...], we_ref[...], preferred_element_type=jnp.float32)
    energy = jnp.tanh(x + hproj)
    s = jnp.dot(energy, v_ref[...], preferred_element_type=jnp.float32)
    out_ref[pl.ds(j * BLK, BLK), :] = s

    @pl.when(j == SPS - 1)
    def _softmax():
        sall = out_ref[...]
        m = jnp.max(sall)
        e = jnp.exp(sall - m)
        out_ref[...] = e / jnp.sum(e)


def kernel(hidden_states, encoder_output, tree_sizes, W, v):
    del tree_sizes
    wd_t = W[:, :H_DEC].T
    we_t = W[:, H_DEC:].T
    out = pl.pallas_call(
        _attn_body,
        grid=(TOTAL // BLK,),
        in_specs=[
            pl.BlockSpec((B, H_DEC), lambda i: (0, 0)),
            pl.BlockSpec((BLK, H_ENC), lambda i: (i, 0)),
            pl.BlockSpec((H_DEC, H_ENC), lambda i: (0, 0)),
            pl.BlockSpec((H_ENC, H_ENC), lambda i: (0, 0)),
            pl.BlockSpec((H_ENC, 1), lambda i: (0, 0)),
        ],
        out_specs=pl.BlockSpec((SEG, 1), lambda i: (i // SPS, 0)),
        out_shape=jax.ShapeDtypeStruct((TOTAL, 1), jnp.float32),
        compiler_params=pltpu.CompilerParams(
            dimension_semantics=("arbitrary",),
        ),
    )(hidden_states, encoder_output, wd_t, we_t, v)
    return out


# speedup vs baseline: 1.3960x; 1.3960x over previous
import jax
import jax.numpy as jnp
from jax import lax
from jax.experimental import pallas as pl
from jax.experimental.pallas import tpu as pltpu

B = 8
H_ENC = 1024
H_DEC = 1024
TOTAL = 16384
SEG = TOTAL // B

_DN_T = (((1,), (1,)), ((), ()))


def _attn_body(hs_ref, enc_ref, w_ref, v_ref, out_ref):
    b = pl.program_id(0)
    h = hs_ref[pl.ds(b, 1), :]
    wd = w_ref[:, :H_DEC]
    we = w_ref[:, H_DEC:]
    hproj = lax.dot_general(h, wd, _DN_T,
                            preferred_element_type=jnp.float32)
    x = lax.dot_general(enc_ref[...], we, _DN_T,
                        preferred_element_type=jnp.float32)
    energy = jnp.tanh(x + hproj)
    s = jnp.dot(energy, v_ref[...], preferred_element_type=jnp.float32)
    m = jnp.max(s)
    e = jnp.exp(s - m)
    out_ref[...] = e / jnp.sum(e)


def kernel(hidden_states, encoder_output, tree_sizes, W, v):
    del tree_sizes
    out = pl.pallas_call(
        _attn_body,
        grid=(B,),
        in_specs=[
            pl.BlockSpec((B, H_DEC), lambda b: (0, 0)),
            pl.BlockSpec((SEG, H_ENC), lambda b: (b, 0)),
            pl.BlockSpec((H_ENC, H_DEC + H_ENC), lambda b: (0, 0)),
            pl.BlockSpec((H_ENC, 1), lambda b: (0, 0)),
        ],
        out_specs=pl.BlockSpec((SEG, 1), lambda b: (b, 0)),
        out_shape=jax.ShapeDtypeStruct((TOTAL, 1), jnp.float32),
        compiler_params=pltpu.CompilerParams(
            dimension_semantics=("parallel",),
        ),
    )(hidden_states, encoder_output, W, v)
    return out


# speedup vs baseline: 1.3978x; 1.0013x over previous
import jax
import jax.numpy as jnp
from jax import lax
from jax.experimental import pallas as pl
from jax.experimental.pallas import tpu as pltpu

B = 8
H_ENC = 1024
H_DEC = 1024
TOTAL = 16384
SEG = TOTAL // B

_DN_T = (((1,), (1,)), ((), ()))


def _attn_body(hs_ref, enc_ref, w_ref, v_ref, out_ref):
    b = pl.program_id(0)
    h = hs_ref[pl.ds(b, 1), :]
    wd = w_ref[:, :H_DEC]
    we = w_ref[:, H_DEC:]
    hproj = lax.dot_general(h, wd, _DN_T,
                            preferred_element_type=jnp.float32)
    x = lax.dot_general(enc_ref[...], we, _DN_T,
                        preferred_element_type=jnp.float32)
    energy = jnp.tanh(x + hproj)
    s = jnp.dot(energy, v_ref[...], preferred_element_type=jnp.float32)
    m = jnp.max(s)
    e = jnp.exp(s - m)
    out_ref[...] = e / jnp.sum(e)


def kernel(hidden_states, encoder_output, tree_sizes, W, v):
    del tree_sizes
    out = pl.pallas_call(
        _attn_body,
        grid=(B,),
        in_specs=[
            pl.BlockSpec((B, H_DEC), lambda b: (0, 0)),
            pl.BlockSpec((SEG, H_ENC), lambda b: (b, 0)),
            pl.BlockSpec((H_ENC, H_DEC + H_ENC), lambda b: (0, 0)),
            pl.BlockSpec((H_ENC, 1), lambda b: (0, 0)),
        ],
        out_specs=pl.BlockSpec((SEG, 1), lambda b: (b, 0)),
        out_shape=jax.ShapeDtypeStruct((TOTAL, 1), jnp.float32),
        compiler_params=pltpu.CompilerParams(
            dimension_semantics=("parallel",),
            vmem_limit_bytes=100 * 1024 * 1024,
        ),
    )(hidden_states, encoder_output, W, v)
    return out


# speedup vs baseline: 1.4060x; 1.0058x over previous
import jax
import jax.numpy as jnp
from jax import lax
from jax.experimental import pallas as pl
from jax.experimental.pallas import tpu as pltpu

B = 8
H_ENC = 1024
H_DEC = 1024
TOTAL = 16384
SEG = TOTAL // B
SEGS_PER_STEP = 2
BLK = SEG * SEGS_PER_STEP

_DN_T = (((1,), (1,)), ((), ()))


def _attn_body(hs_ref, enc_ref, w_ref, v_ref, out_ref):
    i = pl.program_id(0)
    wd = w_ref[:, :H_DEC]
    we = w_ref[:, H_DEC:]
    hproj_all = lax.dot_general(hs_ref[...], wd, _DN_T,
                                preferred_element_type=jnp.float32)
    x = lax.dot_general(enc_ref[...], we, _DN_T,
                        preferred_element_type=jnp.float32)
    iota = lax.broadcasted_iota(jnp.int32, (1, B), 1)
    for k in range(SEGS_PER_STEP):
        onehot = (iota == i * SEGS_PER_STEP + k).astype(jnp.float32)
        row = jnp.dot(onehot, hproj_all,
                      preferred_element_type=jnp.float32)
        energy = jnp.tanh(x[k * SEG:(k + 1) * SEG, :] + row)
        sk = jnp.dot(energy, v_ref[...],
                     preferred_element_type=jnp.float32)
        m = jnp.max(sk)
        e = jnp.exp(sk - m)
        out_ref[k * SEG:(k + 1) * SEG, :] = e / jnp.sum(e)


def kernel(hidden_states, encoder_output, tree_sizes, W, v):
    del tree_sizes
    out = pl.pallas_call(
        _attn_body,
        grid=(TOTAL // BLK,),
        in_specs=[
            pl.BlockSpec((B, H_DEC), lambda i: (0, 0)),
            pl.BlockSpec((BLK, H_ENC), lambda i: (i, 0)),
            pl.BlockSpec((H_ENC, H_DEC + H_ENC), lambda i: (0, 0)),
            pl.BlockSpec((H_ENC, 1), lambda i: (0, 0)),
        ],
        out_specs=pl.BlockSpec((BLK, 1), lambda i: (i, 0)),
        out_shape=jax.ShapeDtypeStruct((TOTAL, 1), jnp.float32),
        compiler_params=pltpu.CompilerParams(
            dimension_semantics=("parallel",),
            vmem_limit_bytes=100 * 1024 * 1024,
        ),
    )(hidden_states, encoder_output, W, v)
    return out


# speedup vs baseline: 1.4335x; 1.0195x over previous
import jax
import jax.numpy as jnp
from jax import lax
from jax.experimental import pallas as pl
from jax.experimental.pallas import tpu as pltpu

B = 8
H_ENC = 1024
H_DEC = 1024
TOTAL = 16384
SEG = TOTAL // B
SEGS_PER_STEP = 2
BLK = SEG * SEGS_PER_STEP

_DN_T = (((1,), (1,)), ((), ()))


def _attn_body(hs_ref, enc_ref, w_ref, v_ref, out_ref):
    i = pl.program_id(0)
    wd = w_ref[:, :H_DEC]
    we = w_ref[:, H_DEC:]
    hproj_all = lax.dot_general(hs_ref[...], wd, _DN_T,
                                preferred_element_type=jnp.float32)
    x = lax.dot_general(enc_ref[...], we, _DN_T,
                        preferred_element_type=jnp.float32)
    iota = lax.broadcasted_iota(jnp.int32, (B, 1), 0)
    for k in range(SEGS_PER_STEP):
        mask = (iota == i * SEGS_PER_STEP + k).astype(jnp.float32)
        row = jnp.sum(hproj_all * mask, axis=0, keepdims=True)
        energy = jnp.tanh(x[k * SEG:(k + 1) * SEG, :] + row)
        sk = jnp.dot(energy, v_ref[...],
                     preferred_element_type=jnp.float32)
        m = jnp.max(sk)
        e = jnp.exp(sk - m)
        out_ref[k * SEG:(k + 1) * SEG, :] = e / jnp.sum(e)


def kernel(hidden_states, encoder_output, tree_sizes, W, v):
    del tree_sizes
    out = pl.pallas_call(
        _attn_body,
        grid=(TOTAL // BLK,),
        in_specs=[
            pl.BlockSpec((B, H_DEC), lambda i: (0, 0)),
            pl.BlockSpec((BLK, H_ENC), lambda i: (i, 0)),
            pl.BlockSpec((H_ENC, H_DEC + H_ENC), lambda i: (0, 0)),
            pl.BlockSpec((H_ENC, 1), lambda i: (0, 0)),
        ],
        out_specs=pl.BlockSpec((BLK, 1), lambda i: (i, 0)),
        out_shape=jax.ShapeDtypeStruct((TOTAL, 1), jnp.float32),
        compiler_params=pltpu.CompilerParams(
            dimension_semantics=("parallel",),
            vmem_limit_bytes=100 * 1024 * 1024,
        ),
    )(hidden_states, encoder_output, W, v)
    return out
